# baseline (speedup 1.0000x reference)
import functools

import jax
import jax.numpy as jnp
from jax import lax
from jax.experimental import pallas as pl
from jax.experimental.pallas import tpu as pltpu
from jax.experimental.pallas import tpu_sc as plsc

NC = 2
NS = 16
K = 80


def _mesh():
    return plsc.VectorSubcoreMesh(core_axis_name="c", subcore_axis_name="s")


def _sc_degree(dst2, zeros1, n_pad):
    ept = dst2.shape[1]

    @functools.partial(
        pl.kernel,
        out_type=jax.ShapeDtypeStruct((NC * NS, n_pad), jnp.float32),
        mesh=_mesh(),
        compiler_params=pltpu.CompilerParams(needs_layout_passes=False),
        scratch_types=[
            pltpu.VMEM((ept,), jnp.int32),
            pltpu.VMEM((n_pad,), jnp.float32),
        ],
    )
    def k(dst_hbm, zero_h, out_hbm, idx_v, hist):
        c = lax.axis_index("c")
        s = lax.axis_index("s")
        w = c * NS + s
        pltpu.sync_copy(zero_h, hist)
        pltpu.sync_copy(dst_hbm.at[w], idx_v)
        ones = jnp.ones((16,), jnp.float32)
        lanes = lax.iota(jnp.int32, 16)

        def body(j, _):
            idx = idx_v[pl.ds(j * 16, 16)]
            for m in range(16):
                plsc.addupdate_scatter(hist, [idx], ones, mask=lanes == m)
            return ()
        lax.fori_loop(0, ept // 16, body, ())
        pltpu.sync_copy(hist, out_hbm.at[w])

    return k(dst2, zeros1)


RING = 3


def _sc_aggregate(table, ig5, dst5, zeros_nd, n_pad, d):
    nch = ig5.shape[2]
    rows_per_tile = n_pad // NS
    R = RING

    @functools.partial(
        pl.kernel,
        out_type=jax.ShapeDtypeStruct((NC, n_pad, d), jnp.float32),
        mesh=_mesh(),
        scratch_types=(
            [pltpu.VMEM((1, K), jnp.int32) for _ in range(2 * R)]
            + [pltpu.VMEM((K, d), jnp.float32) for _ in range(R)]
            + [pltpu.VMEM_SHARED((n_pad, d), jnp.float32)]
            + [pltpu.SemaphoreType.DMA for _ in range(R)]
        ),
    )
    def k(tbl, ig_h, dst_h, zero_h, out_hbm, *scr):
        igb = scr[0:R]
        dsb = scr[R:2 * R]
        rows = scr[2 * R:3 * R]
        acc = scr[3 * R]
        sems = scr[3 * R + 1:]
        c = lax.axis_index("c")
        s = lax.axis_index("s")
        rs = s * rows_per_tile
        pltpu.sync_copy(zero_h.at[pl.ds(rs, rows_per_tile)],
                        acc.at[pl.ds(rs, rows_per_tile)])
        for b in range(R):
            pltpu.sync_copy(ig_h.at[c, s, b], igb[b])
            pltpu.sync_copy(dst_h.at[c, s, b], dsb[b])
        plsc.subcore_barrier()

        for b in range(R):
            pltpu.async_copy(tbl.at[igb[b].at[0]], rows[b], sems[b])

        def step(j, b):
            pltpu.make_async_copy(tbl.at[igb[b].at[0]], rows[b],
                                  sems[b]).wait()
            pltpu.sync_copy(rows[b], acc.at[dsb[b].at[0]], add=True)

            @pl.when(j + R < nch)
            def _():
                pltpu.sync_copy(ig_h.at[c, s, j + R], igb[b])
                pltpu.sync_copy(dst_h.at[c, s, j + R], dsb[b])
                pltpu.async_copy(tbl.at[igb[b].at[0]], rows[b], sems[b])

        def body(p, _):
            for b in range(R):
                step(p * R + b, b)
            return ()
        lax.fori_loop(0, nch // R, body, ())
        for j in range((nch // R) * R, nch):
            step(j, j % R)
        plsc.subcore_barrier()
        pltpu.sync_copy(acc.at[pl.ds(rs, rows_per_tile)],
                        out_hbm.at[c, pl.ds(rs, rows_per_tile)])

    return k(table, ig5, dst5, zeros_nd)


def _sc_decode(zA, zB, sg4, dg4, n_edges, d):
    nch = sg4.shape[2]
    K = sg4.shape[3]
    edges_per_tile = nch * K

    @functools.partial(
        pl.kernel,
        out_type=jax.ShapeDtypeStruct((n_edges, d), jnp.float32),
        mesh=_mesh(),
        scratch_types=[
            pltpu.VMEM((nch, K), jnp.int32),
            pltpu.VMEM((nch, K), jnp.int32),
            pltpu.VMEM((K, d), jnp.float32),
            pltpu.VMEM((K, d), jnp.float32),
            pltpu.VMEM((K, d), jnp.float32),
            pltpu.VMEM((K, d), jnp.float32),
            pltpu.VMEM((K, d), jnp.float32),
            pltpu.SemaphoreType.DMA,
            pltpu.SemaphoreType.DMA,
            pltpu.SemaphoreType.DMA,
            pltpu.SemaphoreType.DMA,
        ],
    )
    def k(za, zb, sg_h, dg_h, out_hbm,
          igv, jgv, a0, a1, b0, b1, sv, sa0, sa1, sb0, sb1):
        c = lax.axis_index("c")
        s = lax.axis_index("s")
        base = (c * NS + s) * edges_per_tile
        pltpu.sync_copy(sg_h.at[c, s], igv)
        pltpu.sync_copy(dg_h.at[c, s], jgv)

        abuf = (a0, a1)
        bbuf = (b0, b1)
        sas = (sa0, sa1)
        sbs = (sb0, sb1)
        pltpu.async_copy(za.at[igv.at[0]], a0, sa0)
        pltpu.async_copy(zb.at[jgv.at[0]], b0, sb0)
        pltpu.async_copy(za.at[igv.at[1]], a1, sa1)
        pltpu.async_copy(zb.at[jgv.at[1]], b1, sb1)

        nv = d // 16

        def body(p, _):
            for b in range(2):
                j = p * 2 + b
                pltpu.make_async_copy(za.at[igv.at[j]], abuf[b], sas[b]).wait()
                pltpu.make_async_copy(zb.at[jgv.at[j]], bbuf[b], sbs[b]).wait()

                def add_row(r, _):
                    for v in range(nv):
                        sl = pl.ds(v * 16, 16)
                        sv[r, sl] = abuf[b][r, sl] + bbuf[b][r, sl]
                    return ()
                lax.fori_loop(0, K, add_row, ())
                pltpu.sync_copy(sv, out_hbm.at[pl.ds(base + j * K, K)])

                @pl.when(j + 2 < nch)
                def _():
                    pltpu.async_copy(za.at[igv.at[j + 2]], abuf[b], sas[b])
                    pltpu.async_copy(zb.at[jgv.at[j + 2]], bbuf[b], sbs[b])
            return ()
        lax.fori_loop(0, nch // 2, body, ())
        if nch % 2:
            b = (nch - 1) % 2
            j = nch - 1
            pltpu.make_async_copy(za.at[igv.at[j]], abuf[b], sas[b]).wait()
            pltpu.make_async_copy(zb.at[jgv.at[j]], bbuf[b], sbs[b]).wait()

            def add_row_t(r, _):
                for v in range(nv):
                    sl = pl.ds(v * 16, 16)
                    sv[r, sl] = abuf[b][r, sl] + bbuf[b][r, sl]
                return ()
            lax.fori_loop(0, K, add_row_t, ())
            pltpu.sync_copy(sv, out_hbm.at[pl.ds(base + j * K, K)])

    return k(zA, zB, sg4, dg4)


def _dis_block(dp_ref, rblk):
    del rblk
    return lax.rsqrt(jnp.sum(dp_ref[...], axis=1) + 1.0)[:, None]


def _tc_y1(x, W1, degp, rblk):
    n, cin = x.shape
    cout = W1.shape[1]

    def body(x_ref, w_ref, dp_ref, y_ref):
        dis = _dis_block(dp_ref, rblk)
        y_ref[...] = dis * jnp.dot(x_ref[...], w_ref[...],
                                   preferred_element_type=jnp.float32)

    return pl.pallas_call(
        body,
        grid=(n // rblk,),
        in_specs=[
            pl.BlockSpec((rblk, cin), lambda i: (i, 0)),
            pl.BlockSpec((cin, cout), lambda i: (0, 0)),
            pl.BlockSpec((rblk, NC * NS), lambda i: (i, 0)),
        ],
        out_specs=pl.BlockSpec((rblk, cout), lambda i: (i, 0)),
        out_shape=jax.ShapeDtypeStruct((n, cout), jnp.float32),
    )(x, W1, degp)


def _tc_layer2(A1, Y1, degp, b1, W2, rblk):
    n = Y1.shape[0]
    d1 = Y1.shape[1]
    d2 = W2.shape[1]

    def body(a_ref, y_ref, dp_ref, b_ref, w_ref, y2_ref):
        dis = _dis_block(dp_ref, rblk)
        agg = jnp.concatenate([a_ref[0], a_ref[1]], axis=-1)
        h = jnp.maximum(dis * (agg + y_ref[...]) + b_ref[...], 0.0)
        y2_ref[...] = dis * jnp.dot(h, w_ref[...],
                                    preferred_element_type=jnp.float32)

    return pl.pallas_call(
        body,
        grid=(n // rblk,),
        in_specs=[
            pl.BlockSpec((2, rblk, d1 // 2), lambda i: (0, i, 0)),
            pl.BlockSpec((rblk, d1), lambda i: (i, 0)),
            pl.BlockSpec((rblk, NC * NS), lambda i: (i, 0)),
            pl.BlockSpec((1, d1), lambda i: (0, 0)),
            pl.BlockSpec((d1, d2), lambda i: (0, 0)),
        ],
        out_specs=pl.BlockSpec((rblk, d2), lambda i: (i, 0)),
        out_shape=jax.ShapeDtypeStruct((n, d2), jnp.float32),
    )(A1, Y1, degp, b1, W2)


def _tc_z_proj(A2, Y2, degp, b2, fcA, fcB, rblk):
    n, d2 = Y2.shape
    dp = fcA.shape[1]

    def body(a_ref, y_ref, dp_ref, b_ref, wa_ref, wb_ref, za_ref, zb_ref):
        dis = _dis_block(dp_ref, rblk)
        agg = a_ref[0] + a_ref[1]
        z = dis * (agg + y_ref[...]) + b_ref[...]
        za_ref[...] = jnp.dot(z, wa_ref[...], preferred_element_type=jnp.float32)
        zb_ref[...] = jnp.dot(z, wb_ref[...], preferred_element_type=jnp.float32)

    return pl.pallas_call(
        body,
        grid=(n // rblk,),
        in_specs=[
            pl.BlockSpec((2, rblk, d2), lambda i: (0, i, 0)),
            pl.BlockSpec((rblk, d2), lambda i: (i, 0)),
            pl.BlockSpec((rblk, NC * NS), lambda i: (i, 0)),
            pl.BlockSpec((1, d2), lambda i: (0, 0)),
            pl.BlockSpec((d2, dp), lambda i: (0, 0)),
            pl.BlockSpec((d2, dp), lambda i: (0, 0)),
        ],
        out_specs=[
            pl.BlockSpec((rblk, dp), lambda i: (i, 0)),
            pl.BlockSpec((rblk, dp), lambda i: (i, 0)),
        ],
        out_shape=[
            jax.ShapeDtypeStruct((n, dp), jnp.float32),
            jax.ShapeDtypeStruct((n, dp), jnp.float32),
        ],
    )(A2, Y2, degp, b2, fcA, fcB)


def _tc_mlp(S, fc1_b, fc2_W, fc2_b, fc3_W, fc3_b, fc4_W, fc4_b, eblk):
    e, dh = S.shape

    bf = jnp.bfloat16

    def body(s_ref, b1_ref, w2_ref, b2_ref, w3_ref, b3_ref, w4_ref, b4_ref,
             o_ref):
        v = jnp.maximum(s_ref[...] + b1_ref[...], 0.0)
        v = jnp.maximum(jnp.dot(v.astype(bf), w2_ref[...].astype(bf),
                                preferred_element_type=jnp.float32)
                        + b2_ref[...], 0.0)
        v = jnp.maximum(jnp.dot(v.astype(bf), w3_ref[...].astype(bf),
                                preferred_element_type=jnp.float32)
                        + b3_ref[...], 0.0)
        o_ref[...] = jnp.dot(v.astype(bf), w4_ref[...].astype(bf),
                             preferred_element_type=jnp.float32) + b4_ref[...]

    return pl.pallas_call(
        body,
        grid=(e // eblk,),
        in_specs=[
            pl.BlockSpec((eblk, dh), lambda i: (i, 0)),
            pl.BlockSpec((1, dh), lambda i: (0, 0)),
            pl.BlockSpec(fc2_W.shape, lambda i: (0, 0)),
            pl.BlockSpec((1, fc2_W.shape[1]), lambda i: (0, 0)),
            pl.BlockSpec(fc3_W.shape, lambda i: (0, 0)),
            pl.BlockSpec((1, fc3_W.shape[1]), lambda i: (0, 0)),
            pl.BlockSpec(fc4_W.shape, lambda i: (0, 0)),
            pl.BlockSpec((1, 1), lambda i: (0, 0)),
        ],
        out_specs=pl.BlockSpec((eblk, 1), lambda i: (i, 0)),
        out_shape=jax.ShapeDtypeStruct((e, 1), jnp.float32),
    )(S, fc1_b, fc2_W, fc2_b, fc3_W, fc3_b, fc4_W, fc4_b)


def kernel(x, edge_index, W1, b1, W2, b2,
           fc1_W, fc1_b, fc2_W, fc2_b, fc3_W, fc3_b, fc4_W, fc4_b):
    n, cin = x.shape
    e = edge_index.shape[1]
    nw = NC * NS

    ei = edge_index.astype(jnp.int32)
    src, dst = ei[0], ei[1]

    n_pad = -(-n // (NS * 8)) * (NS * 8)

    ig1 = jnp.stack([2 * src, 2 * src + 1]).reshape(NC, NS, -1, 1, K)
    dst1 = jnp.broadcast_to(dst.reshape(1, NS, -1, 1, K),
                            (NC, NS, e // (NS * K), 1, K))
    src2 = src.reshape(NC, NS, -1, 1, K)
    dst2 = dst.reshape(NC, NS, -1, 1, K)
    sg4 = src.reshape(NC, NS, -1, K)
    dg4 = dst.reshape(NC, NS, -1, K)
    dst_deg = dst.reshape(nw, -1)

    zeros1 = jnp.zeros((n_pad,), jnp.float32)
    zeros128 = jnp.zeros((n_pad, W1.shape[1] // 2), jnp.float32)

    degp = _sc_degree(dst_deg, zeros1, n_pad).T

    Y1 = _tc_y1(x, W1, degp, rblk=1000)
    A1 = _sc_aggregate(Y1.reshape(2 * n, -1), ig1, dst1, zeros128,
                       n_pad, W1.shape[1] // 2)

    Y2 = _tc_layer2(A1, Y1, degp, b1.reshape(1, -1), W2, rblk=1000)
    A2 = _sc_aggregate(Y2, src2, dst2, zeros128,
                       n_pad, W2.shape[1])

    fcA = fc1_W[:W2.shape[1]]
    fcB = fc1_W[W2.shape[1]:]
    zA, zB = _tc_z_proj(A2, Y2, degp, b2.reshape(1, -1), fcA, fcB, rblk=1000)

    km = 2
    eh = e // km
    outs = []
    for i in range(km):
        sgi = src[i * eh:(i + 1) * eh].reshape(NC, NS, -1, 40)
        dgi = dst[i * eh:(i + 1) * eh].reshape(NC, NS, -1, 40)
        Si = _sc_decode(zA, zB, sgi, dgi, eh, fc1_W.shape[1])
        outs.append(_tc_mlp(Si, fc1_b.reshape(1, -1), fc2_W,
                            fc2_b.reshape(1, -1), fc3_W, fc3_b.reshape(1, -1),
                            fc4_W, fc4_b.reshape(1, -1), eblk=4000))
    return jnp.concatenate(outs).reshape(-1)


# speedup vs baseline: 1.0387x; 1.0387x over previous
import functools

import jax
import jax.numpy as jnp
from jax import lax
from jax.experimental import pallas as pl
from jax.experimental.pallas import tpu as pltpu
from jax.experimental.pallas import tpu_sc as plsc

NC = 2
NS = 16
K = 80


def _mesh():
    return plsc.VectorSubcoreMesh(core_axis_name="c", subcore_axis_name="s")


def _sc_degree(dst2, zeros1, n_pad):
    ept = dst2.shape[1]

    @functools.partial(
        pl.kernel,
        out_type=jax.ShapeDtypeStruct((NC * NS, n_pad), jnp.float32),
        mesh=_mesh(),
        compiler_params=pltpu.CompilerParams(needs_layout_passes=False),
        scratch_types=[
            pltpu.VMEM((ept,), jnp.int32),
            pltpu.VMEM((n_pad,), jnp.float32),
        ],
    )
    def k(dst_hbm, zero_h, out_hbm, idx_v, hist):
        c = lax.axis_index("c")
        s = lax.axis_index("s")
        w = c * NS + s
        pltpu.sync_copy(zero_h, hist)
        pltpu.sync_copy(dst_hbm.at[w], idx_v)
        ones = jnp.ones((16,), jnp.float32)
        lanes = lax.iota(jnp.int32, 16)

        def body(j, _):
            idx = idx_v[pl.ds(j * 16, 16)]
            for m in range(16):
                plsc.addupdate_scatter(hist, [idx], ones, mask=lanes == m)
            return ()
        lax.fori_loop(0, ept // 16, body, ())
        pltpu.sync_copy(hist, out_hbm.at[w])

    return k(dst2, zeros1)


RING = 3


def _sc_aggregate(table, ig5, dst5, zeros_nd, n_pad, d):
    nch = ig5.shape[2]
    rows_per_tile = n_pad // NS
    R = RING

    @functools.partial(
        pl.kernel,
        out_type=jax.ShapeDtypeStruct((NC, n_pad, d), jnp.float32),
        mesh=_mesh(),
        scratch_types=(
            [pltpu.VMEM((1, K), jnp.int32) for _ in range(2 * R)]
            + [pltpu.VMEM((K, d), jnp.float32) for _ in range(R)]
            + [pltpu.VMEM_SHARED((n_pad, d), jnp.float32)]
            + [pltpu.SemaphoreType.DMA for _ in range(R)]
        ),
    )
    def k(tbl, ig_h, dst_h, zero_h, out_hbm, *scr):
        igb = scr[0:R]
        dsb = scr[R:2 * R]
        rows = scr[2 * R:3 * R]
        acc = scr[3 * R]
        sems = scr[3 * R + 1:]
        c = lax.axis_index("c")
        s = lax.axis_index("s")
        rs = s * rows_per_tile
        pltpu.sync_copy(zero_h.at[pl.ds(rs, rows_per_tile)],
                        acc.at[pl.ds(rs, rows_per_tile)])
        for b in range(R):
            pltpu.sync_copy(ig_h.at[c, s, b], igb[b])
            pltpu.sync_copy(dst_h.at[c, s, b], dsb[b])
        plsc.subcore_barrier()

        for b in range(R):
            pltpu.async_copy(tbl.at[igb[b].at[0]], rows[b], sems[b])

        def step(j, b):
            pltpu.make_async_copy(tbl.at[igb[b].at[0]], rows[b],
                                  sems[b]).wait()
            pltpu.sync_copy(rows[b], acc.at[dsb[b].at[0]], add=True)

            @pl.when(j + R < nch)
            def _():
                pltpu.sync_copy(ig_h.at[c, s, j + R], igb[b])
                pltpu.sync_copy(dst_h.at[c, s, j + R], dsb[b])
                pltpu.async_copy(tbl.at[igb[b].at[0]], rows[b], sems[b])

        def body(p, _):
            for b in range(R):
                step(p * R + b, b)
            return ()
        lax.fori_loop(0, nch // R, body, ())
        for j in range((nch // R) * R, nch):
            step(j, j % R)
        plsc.subcore_barrier()
        pltpu.sync_copy(acc.at[pl.ds(rs, rows_per_tile)],
                        out_hbm.at[c, pl.ds(rs, rows_per_tile)])

    return k(table, ig5, dst5, zeros_nd)


def _sc_decode(zA, zB, sg4, dg4, n_edges, d):
    nch = sg4.shape[2]
    K = sg4.shape[3]
    edges_per_tile = nch * K

    @functools.partial(
        pl.kernel,
        out_type=jax.ShapeDtypeStruct((n_edges, d), jnp.float32),
        mesh=_mesh(),
        scratch_types=[
            pltpu.VMEM((nch, K), jnp.int32),
            pltpu.VMEM((nch, K), jnp.int32),
            pltpu.VMEM((K, d), jnp.float32),
            pltpu.VMEM((K, d), jnp.float32),
            pltpu.VMEM((K, d), jnp.float32),
            pltpu.VMEM((K, d), jnp.float32),
            pltpu.VMEM((K, d), jnp.float32),
            pltpu.SemaphoreType.DMA,
            pltpu.SemaphoreType.DMA,
            pltpu.SemaphoreType.DMA,
            pltpu.SemaphoreType.DMA,
        ],
    )
    def k(za, zb, sg_h, dg_h, out_hbm,
          igv, jgv, a0, a1, b0, b1, sv, sa0, sa1, sb0, sb1):
        c = lax.axis_index("c")
        s = lax.axis_index("s")
        base = (c * NS + s) * edges_per_tile
        pltpu.sync_copy(sg_h.at[c, s], igv)
        pltpu.sync_copy(dg_h.at[c, s], jgv)

        abuf = (a0, a1)
        bbuf = (b0, b1)
        sas = (sa0, sa1)
        sbs = (sb0, sb1)
        pltpu.async_copy(za.at[igv.at[0]], a0, sa0)
        pltpu.async_copy(zb.at[jgv.at[0]], b0, sb0)
        pltpu.async_copy(za.at[igv.at[1]], a1, sa1)
        pltpu.async_copy(zb.at[jgv.at[1]], b1, sb1)

        nv = d // 16

        def body(p, _):
            for b in range(2):
                j = p * 2 + b
                pltpu.make_async_copy(za.at[igv.at[j]], abuf[b], sas[b]).wait()
                pltpu.make_async_copy(zb.at[jgv.at[j]], bbuf[b], sbs[b]).wait()

                def add_row(r, _):
                    for v in range(nv):
                        sl = pl.ds(v * 16, 16)
                        sv[r, sl] = abuf[b][r, sl] + bbuf[b][r, sl]
                    return ()
                lax.fori_loop(0, K, add_row, ())
                pltpu.sync_copy(sv, out_hbm.at[pl.ds(base + j * K, K)])

                @pl.when(j + 2 < nch)
                def _():
                    pltpu.async_copy(za.at[igv.at[j + 2]], abuf[b], sas[b])
                    pltpu.async_copy(zb.at[jgv.at[j + 2]], bbuf[b], sbs[b])
            return ()
        lax.fori_loop(0, nch // 2, body, ())
        if nch % 2:
            b = (nch - 1) % 2
            j = nch - 1
            pltpu.make_async_copy(za.at[igv.at[j]], abuf[b], sas[b]).wait()
            pltpu.make_async_copy(zb.at[jgv.at[j]], bbuf[b], sbs[b]).wait()

            def add_row_t(r, _):
                for v in range(nv):
                    sl = pl.ds(v * 16, 16)
                    sv[r, sl] = abuf[b][r, sl] + bbuf[b][r, sl]
                return ()
            lax.fori_loop(0, K, add_row_t, ())
            pltpu.sync_copy(sv, out_hbm.at[pl.ds(base + j * K, K)])

    return k(zA, zB, sg4, dg4)


def _dis_block(dp_ref, rblk):
    del rblk
    return lax.rsqrt(jnp.sum(dp_ref[...], axis=1) + 1.0)[:, None]


def _tc_y1(x, W1, degp, rblk):
    n, cin = x.shape
    cout = W1.shape[1]

    def body(x_ref, w_ref, dp_ref, y_ref):
        dis = _dis_block(dp_ref, rblk)
        y_ref[...] = dis * jnp.dot(x_ref[...], w_ref[...],
                                   preferred_element_type=jnp.float32)

    return pl.pallas_call(
        body,
        grid=(n // rblk,),
        in_specs=[
            pl.BlockSpec((rblk, cin), lambda i: (i, 0)),
            pl.BlockSpec((cin, cout), lambda i: (0, 0)),
            pl.BlockSpec((rblk, NC * NS), lambda i: (i, 0)),
        ],
        out_specs=pl.BlockSpec((rblk, cout), lambda i: (i, 0)),
        out_shape=jax.ShapeDtypeStruct((n, cout), jnp.float32),
    )(x, W1, degp)


def _tc_layer2(A1, Y1, degp, b1, W2, rblk):
    n = Y1.shape[0]
    d1 = Y1.shape[1]
    d2 = W2.shape[1]

    def body(a_ref, y_ref, dp_ref, b_ref, w_ref, y2_ref):
        dis = _dis_block(dp_ref, rblk)
        agg = jnp.concatenate([a_ref[0], a_ref[1]], axis=-1)
        h = jnp.maximum(dis * (agg + y_ref[...]) + b_ref[...], 0.0)
        y2_ref[...] = dis * jnp.dot(h, w_ref[...],
                                    preferred_element_type=jnp.float32)

    return pl.pallas_call(
        body,
        grid=(n // rblk,),
        in_specs=[
            pl.BlockSpec((2, rblk, d1 // 2), lambda i: (0, i, 0)),
            pl.BlockSpec((rblk, d1), lambda i: (i, 0)),
            pl.BlockSpec((rblk, NC * NS), lambda i: (i, 0)),
            pl.BlockSpec((1, d1), lambda i: (0, 0)),
            pl.BlockSpec((d1, d2), lambda i: (0, 0)),
        ],
        out_specs=pl.BlockSpec((rblk, d2), lambda i: (i, 0)),
        out_shape=jax.ShapeDtypeStruct((n, d2), jnp.float32),
    )(A1, Y1, degp, b1, W2)


def _tc_z_proj(A2, Y2, degp, b2, fcA, fcB, rblk):
    n, d2 = Y2.shape
    dp = fcA.shape[1]

    def body(a_ref, y_ref, dp_ref, b_ref, wa_ref, wb_ref, za_ref, zb_ref):
        dis = _dis_block(dp_ref, rblk)
        agg = a_ref[0] + a_ref[1]
        z = dis * (agg + y_ref[...]) + b_ref[...]
        za_ref[...] = jnp.dot(z, wa_ref[...], preferred_element_type=jnp.float32)
        zb_ref[...] = jnp.dot(z, wb_ref[...], preferred_element_type=jnp.float32)

    return pl.pallas_call(
        body,
        grid=(n // rblk,),
        in_specs=[
            pl.BlockSpec((2, rblk, d2), lambda i: (0, i, 0)),
            pl.BlockSpec((rblk, d2), lambda i: (i, 0)),
            pl.BlockSpec((rblk, NC * NS), lambda i: (i, 0)),
            pl.BlockSpec((1, d2), lambda i: (0, 0)),
            pl.BlockSpec((d2, dp), lambda i: (0, 0)),
            pl.BlockSpec((d2, dp), lambda i: (0, 0)),
        ],
        out_specs=[
            pl.BlockSpec((rblk, dp), lambda i: (i, 0)),
            pl.BlockSpec((rblk, dp), lambda i: (i, 0)),
        ],
        out_shape=[
            jax.ShapeDtypeStruct((n, dp), jnp.float32),
            jax.ShapeDtypeStruct((n, dp), jnp.float32),
        ],
    )(A2, Y2, degp, b2, fcA, fcB)


def _tc_mlp(S, fc1_b, fc2_W, fc2_b, fc3_W, fc3_b, fc4_W, fc4_b, eblk):
    e, dh = S.shape

    bf = jnp.bfloat16

    def body(s_ref, b1_ref, w2_ref, b2_ref, w3_ref, b3_ref, w4_ref, b4_ref,
             o_ref):
        v = jnp.maximum(s_ref[...] + b1_ref[...], 0.0)
        v = jnp.maximum(jnp.dot(v.astype(bf), w2_ref[...].astype(bf),
                                preferred_element_type=jnp.float32)
                        + b2_ref[...], 0.0)
        v = jnp.maximum(jnp.dot(v.astype(bf), w3_ref[...].astype(bf),
                                preferred_element_type=jnp.float32)
                        + b3_ref[...], 0.0)
        o_ref[...] = jnp.dot(v.astype(bf), w4_ref[...].astype(bf),
                             preferred_element_type=jnp.float32) + b4_ref[...]

    return pl.pallas_call(
        body,
        grid=(e // eblk,),
        in_specs=[
            pl.BlockSpec((eblk, dh), lambda i: (i, 0)),
            pl.BlockSpec((1, dh), lambda i: (0, 0)),
            pl.BlockSpec(fc2_W.shape, lambda i: (0, 0)),
            pl.BlockSpec((1, fc2_W.shape[1]), lambda i: (0, 0)),
            pl.BlockSpec(fc3_W.shape, lambda i: (0, 0)),
            pl.BlockSpec((1, fc3_W.shape[1]), lambda i: (0, 0)),
            pl.BlockSpec(fc4_W.shape, lambda i: (0, 0)),
            pl.BlockSpec((1, 1), lambda i: (0, 0)),
        ],
        out_specs=pl.BlockSpec((eblk, 1), lambda i: (i, 0)),
        out_shape=jax.ShapeDtypeStruct((e, 1), jnp.float32),
    )(S, fc1_b, fc2_W, fc2_b, fc3_W, fc3_b, fc4_W, fc4_b)


def kernel(x, edge_index, W1, b1, W2, b2,
           fc1_W, fc1_b, fc2_W, fc2_b, fc3_W, fc3_b, fc4_W, fc4_b):
    n, cin = x.shape
    e = edge_index.shape[1]
    nw = NC * NS

    ei = edge_index.astype(jnp.int32)
    src, dst = ei[0], ei[1]

    n_pad = -(-n // (NS * 8)) * (NS * 8)

    ig1 = jnp.stack([2 * src, 2 * src + 1]).reshape(NC, NS, -1, 1, K)
    dst1 = jnp.broadcast_to(dst.reshape(1, NS, -1, 1, K),
                            (NC, NS, e // (NS * K), 1, K))
    src2 = src.reshape(NC, NS, -1, 1, K)
    dst2 = dst.reshape(NC, NS, -1, 1, K)
    sg4 = src.reshape(NC, NS, -1, K)
    dg4 = dst.reshape(NC, NS, -1, K)
    dst_deg = dst.reshape(nw, -1)

    zeros1 = jnp.zeros((n_pad,), jnp.float32)
    zeros128 = jnp.zeros((n_pad, W1.shape[1] // 2), jnp.float32)

    degp = _sc_degree(dst_deg, zeros1, n_pad).T

    Y1 = _tc_y1(x, W1, degp, rblk=1000)
    A1 = _sc_aggregate(Y1.reshape(2 * n, -1), ig1, dst1, zeros128,
                       n_pad, W1.shape[1] // 2)

    Y2 = _tc_layer2(A1, Y1, degp, b1.reshape(1, -1), W2, rblk=1000)
    A2 = _sc_aggregate(Y2, src2, dst2, zeros128,
                       n_pad, W2.shape[1])

    fcA = fc1_W[:W2.shape[1]]
    fcB = fc1_W[W2.shape[1]:]
    zA, zB = _tc_z_proj(A2, Y2, degp, b2.reshape(1, -1), fcA, fcB, rblk=1000)

    S = _sc_decode(zA, zB, sg4, dg4, e, fc1_W.shape[1])

    out = _tc_mlp(S, fc1_b.reshape(1, -1), fc2_W, fc2_b.reshape(1, -1),
                  fc3_W, fc3_b.reshape(1, -1), fc4_W,
                  fc4_b.reshape(1, -1), eblk=4000)
    return out.reshape(-1)


# speedup vs baseline: 1.1684x; 1.1248x over previous
import functools

import jax
import jax.numpy as jnp
from jax import lax
from jax.experimental import pallas as pl
from jax.experimental.pallas import tpu as pltpu
from jax.experimental.pallas import tpu_sc as plsc

NC = 2
NS = 16
K = 80


def _mesh():
    return plsc.VectorSubcoreMesh(core_axis_name="c", subcore_axis_name="s")


def _sc_degree(dst_flat, zeros1, n_pad):
    ept = dst_flat.shape[0] // (NC * NS)

    @functools.partial(
        pl.kernel,
        out_type=jax.ShapeDtypeStruct((NC * NS, n_pad), jnp.float32),
        mesh=_mesh(),
        compiler_params=pltpu.CompilerParams(needs_layout_passes=False),
        scratch_types=[
            pltpu.VMEM((ept,), jnp.int32),
            pltpu.VMEM((n_pad,), jnp.float32),
        ],
    )
    def k(dst_hbm, zero_h, out_hbm, idx_v, hist):
        c = lax.axis_index("c")
        s = lax.axis_index("s")
        w = c * NS + s
        pltpu.sync_copy(zero_h, hist)
        pltpu.sync_copy(dst_hbm.at[pl.ds(w * ept, ept)], idx_v)
        ones = jnp.ones((16,), jnp.float32)
        lanes = lax.iota(jnp.int32, 16)

        def body(j, _):
            idx = idx_v[pl.ds(j * 16, 16)]
            for m in range(16):
                plsc.addupdate_scatter(hist, [idx], ones, mask=lanes == m)
            return ()
        lax.fori_loop(0, ept // 16, body, ())
        pltpu.sync_copy(hist, out_hbm.at[w])

    return k(dst_flat, zeros1)


RING = 3


def _sc_aggregate(table, src_flat, dst_flat, zeros_nd, n_pad, d, feat_split):
    e = src_flat.shape[0]
    ept = e // NS if feat_split else e // (NC * NS)
    nch = ept // K
    coeff = table.shape[0] // 2 if feat_split else 0
    rows_per_tile = n_pad // NS
    R = RING

    @functools.partial(
        pl.kernel,
        out_type=jax.ShapeDtypeStruct((NC, n_pad, d), jnp.float32),
        mesh=_mesh(),
        compiler_params=pltpu.CompilerParams(needs_layout_passes=False),
        scratch_types=(
            [pltpu.VMEM((K,), jnp.int32) for _ in range(2 * R)]
            + [pltpu.VMEM((K, d), jnp.float32) for _ in range(R)]
            + [pltpu.VMEM_SHARED((n_pad, d), jnp.float32)]
            + [pltpu.SemaphoreType.DMA for _ in range(R)]
        ),
    )
    def k(tbl, src_h, dst_h, zero_h, out_hbm, *scr):
        igb = scr[0:R]
        dsb = scr[R:2 * R]
        rows = scr[2 * R:3 * R]
        acc = scr[3 * R]
        sems = scr[3 * R + 1:]
        c = lax.axis_index("c")
        s = lax.axis_index("s")
        rs = s * rows_per_tile
        eoff = (s if feat_split else c * NS + s) * ept
        off = c * coeff
        pltpu.sync_copy(zero_h.at[pl.ds(rs, rows_per_tile)],
                        acc.at[pl.ds(rs, rows_per_tile)])

        def load_idx(j, b):
            pltpu.sync_copy(src_h.at[pl.ds(eoff + j * K, K)], igb[b])
            pltpu.sync_copy(dst_h.at[pl.ds(eoff + j * K, K)], dsb[b])
            if feat_split:
                for v in range(K // 16):
                    sl = pl.ds(v * 16, 16)
                    igb[b][sl] = igb[b][sl] + off

        for b in range(R):
            load_idx(b, b)
        plsc.subcore_barrier()

        for b in range(R):
            pltpu.async_copy(tbl.at[igb[b]], rows[b], sems[b])

        def step(j, b):
            pltpu.make_async_copy(tbl.at[igb[b]], rows[b], sems[b]).wait()
            pltpu.sync_copy(rows[b], acc.at[dsb[b]], add=True)

            def refill():
                load_idx(j + R, b)
                pltpu.async_copy(tbl.at[igb[b]], rows[b], sems[b])

            if isinstance(j, int):
                if j + R < nch:
                    refill()
            else:
                pl.when(j + R < nch)(refill)

        def body(p, _):
            for b in range(R):
                step(p * R + b, b)
            return ()
        lax.fori_loop(0, nch // R, body, ())
        for j in range((nch // R) * R, nch):
            step(j, j % R)
        plsc.subcore_barrier()
        pltpu.sync_copy(acc.at[pl.ds(rs, rows_per_tile)],
                        out_hbm.at[c, pl.ds(rs, rows_per_tile)])

    return k(table, src_flat, dst_flat, zeros_nd)


def _sc_decode(zA, zB, src_flat, dst_flat, d):
    e = src_flat.shape[0]
    ept = e // (NC * NS)
    nch = ept // K

    @functools.partial(
        pl.kernel,
        out_type=jax.ShapeDtypeStruct((e, d), jnp.float32),
        mesh=_mesh(),
        scratch_types=(
            [pltpu.VMEM((K,), jnp.int32) for _ in range(4)]
            + [pltpu.VMEM((K, d), jnp.float32) for _ in range(6)]
            + [pltpu.SemaphoreType.DMA for _ in range(6)]
        ),
    )
    def k(za, zb, src_h, dst_h, out_hbm, *scr):
        igv = scr[0:2]
        jgv = scr[2:4]
        abuf = scr[4:6]
        bbuf = scr[6:8]
        svs = scr[8:10]
        sas = scr[10:12]
        sbs = scr[12:14]
        sws = scr[14:16]
        c = lax.axis_index("c")
        s = lax.axis_index("s")
        base = (c * NS + s) * ept

        def load_idx(j, b):
            pltpu.sync_copy(src_h.at[pl.ds(base + j * K, K)], igv[b])
            pltpu.sync_copy(dst_h.at[pl.ds(base + j * K, K)], jgv[b])

        for b in range(2):
            load_idx(b, b)
            pltpu.async_copy(za.at[igv[b]], abuf[b], sas[b])
            pltpu.async_copy(zb.at[jgv[b]], bbuf[b], sbs[b])

        nv = d // 16

        def step(j, b):
            pltpu.make_async_copy(za.at[igv[b]], abuf[b], sas[b]).wait()
            pltpu.make_async_copy(zb.at[jgv[b]], bbuf[b], sbs[b]).wait()

            def drain_write():
                pltpu.make_async_copy(
                    svs[b], out_hbm.at[pl.ds(base, K)], sws[b]).wait()

            if isinstance(j, int):
                if j >= 2:
                    drain_write()
            else:
                pl.when(j >= 2)(drain_write)

            def add_row(r, _):
                for v in range(nv):
                    sl = pl.ds(v * 16, 16)
                    svs[b][r, sl] = abuf[b][r, sl] + bbuf[b][r, sl]
                return ()
            lax.fori_loop(0, K, add_row, ())
            pltpu.async_copy(svs[b], out_hbm.at[pl.ds(base + j * K, K)],
                             sws[b])

            def refill():
                load_idx(j + 2, b)
                pltpu.async_copy(za.at[igv[b]], abuf[b], sas[b])
                pltpu.async_copy(zb.at[jgv[b]], bbuf[b], sbs[b])

            if isinstance(j, int):
                if j + 2 < nch:
                    refill()
            else:
                pl.when(j + 2 < nch)(refill)

        def body(p, _):
            for b in range(2):
                step(p * 2 + b, b)
            return ()
        lax.fori_loop(0, nch // 2, body, ())
        for j in range((nch // 2) * 2, nch):
            step(j, j % 2)
        for b in range(2):
            pltpu.make_async_copy(svs[b], out_hbm.at[pl.ds(base, K)],
                                  sws[b]).wait()

    return k(zA, zB, src_flat, dst_flat)


def _dis_block(dp_ref, rblk):
    del rblk
    return lax.rsqrt(jnp.sum(dp_ref[...], axis=1) + 1.0)[:, None]


def _tc_y1(x, W1, degp, rblk):
    n, cin = x.shape
    cout = W1.shape[1]
    half = cout // 2

    def body(x_ref, w_ref, dp_ref, y_ref):
        dis = _dis_block(dp_ref, rblk)
        y = dis * jnp.dot(x_ref[...], w_ref[...],
                          preferred_element_type=jnp.float32)
        y_ref[0, ...] = y[:, :half]
        y_ref[1, ...] = y[:, half:]

    return pl.pallas_call(
        body,
        grid=(n // rblk,),
        in_specs=[
            pl.BlockSpec((rblk, cin), lambda i: (i, 0)),
            pl.BlockSpec((cin, cout), lambda i: (0, 0)),
            pl.BlockSpec((rblk, NC * NS), lambda i: (i, 0)),
        ],
        out_specs=pl.BlockSpec((2, rblk, half), lambda i: (0, i, 0)),
        out_shape=jax.ShapeDtypeStruct((2, n, half), jnp.float32),
    )(x, W1, degp)


def _tc_layer2(A1, Y1p, degp, b1, W2, rblk):
    n = Y1p.shape[1]
    d1 = 2 * Y1p.shape[2]
    d2 = W2.shape[1]

    def body(a_ref, y_ref, dp_ref, b_ref, w_ref, y2_ref):
        dis = _dis_block(dp_ref, rblk)
        agg = jnp.concatenate([a_ref[0], a_ref[1]], axis=-1)
        y = jnp.concatenate([y_ref[0], y_ref[1]], axis=-1)
        h = jnp.maximum(dis * (agg + y) + b_ref[...], 0.0)
        y2_ref[...] = dis * jnp.dot(h, w_ref[...],
                                    preferred_element_type=jnp.float32)

    return pl.pallas_call(
        body,
        grid=(n // rblk,),
        in_specs=[
            pl.BlockSpec((2, rblk, d1 // 2), lambda i: (0, i, 0)),
            pl.BlockSpec((2, rblk, d1 // 2), lambda i: (0, i, 0)),
            pl.BlockSpec((rblk, NC * NS), lambda i: (i, 0)),
            pl.BlockSpec((1, d1), lambda i: (0, 0)),
            pl.BlockSpec((d1, d2), lambda i: (0, 0)),
        ],
        out_specs=pl.BlockSpec((rblk, d2), lambda i: (i, 0)),
        out_shape=jax.ShapeDtypeStruct((n, d2), jnp.float32),
    )(A1, Y1p, degp, b1, W2)


def _tc_z_proj(A2, Y2, degp, b2, fcA, fcB, rblk):
    n, d2 = Y2.shape
    dp = fcA.shape[1]

    def body(a_ref, y_ref, dp_ref, b_ref, wa_ref, wb_ref, za_ref, zb_ref):
        dis = _dis_block(dp_ref, rblk)
        agg = a_ref[0] + a_ref[1]
        z = dis * (agg + y_ref[...]) + b_ref[...]
        za_ref[...] = jnp.dot(z, wa_ref[...], preferred_element_type=jnp.float32)
        zb_ref[...] = jnp.dot(z, wb_ref[...], preferred_element_type=jnp.float32)

    return pl.pallas_call(
        body,
        grid=(n // rblk,),
        in_specs=[
            pl.BlockSpec((2, rblk, d2), lambda i: (0, i, 0)),
            pl.BlockSpec((rblk, d2), lambda i: (i, 0)),
            pl.BlockSpec((rblk, NC * NS), lambda i: (i, 0)),
            pl.BlockSpec((1, d2), lambda i: (0, 0)),
            pl.BlockSpec((d2, dp), lambda i: (0, 0)),
            pl.BlockSpec((d2, dp), lambda i: (0, 0)),
        ],
        out_specs=[
            pl.BlockSpec((rblk, dp), lambda i: (i, 0)),
            pl.BlockSpec((rblk, dp), lambda i: (i, 0)),
        ],
        out_shape=[
            jax.ShapeDtypeStruct((n, dp), jnp.float32),
            jax.ShapeDtypeStruct((n, dp), jnp.float32),
        ],
    )(A2, Y2, degp, b2, fcA, fcB)


def _tc_mlp(S, fc1_b, fc2_W, fc2_b, fc3_W, fc3_b, fc4_W, fc4_b, eblk):
    e, dh = S.shape

    bf = jnp.bfloat16

    def body(s_ref, b1_ref, w2_ref, b2_ref, w3_ref, b3_ref, w4_ref, b4_ref,
             o_ref):
        v = jnp.maximum(s_ref[...] + b1_ref[...], 0.0)
        v = jnp.maximum(jnp.dot(v.astype(bf), w2_ref[...].astype(bf),
                                preferred_element_type=jnp.float32)
                        + b2_ref[...], 0.0)
        v = jnp.maximum(jnp.dot(v.astype(bf), w3_ref[...].astype(bf),
                                preferred_element_type=jnp.float32)
                        + b3_ref[...], 0.0)
        o = jnp.sum(v * w4_ref[...].reshape(1, -1), axis=1) + b4_ref[0, 0]
        o_ref[...] = o.reshape(1, eblk // 128, 128)

    return pl.pallas_call(
        body,
        grid=(e // eblk,),
        in_specs=[
            pl.BlockSpec((eblk, dh), lambda i: (i, 0)),
            pl.BlockSpec((1, dh), lambda i: (0, 0)),
            pl.BlockSpec(fc2_W.shape, lambda i: (0, 0)),
            pl.BlockSpec((1, fc2_W.shape[1]), lambda i: (0, 0)),
            pl.BlockSpec(fc3_W.shape, lambda i: (0, 0)),
            pl.BlockSpec((1, fc3_W.shape[1]), lambda i: (0, 0)),
            pl.BlockSpec(fc4_W.shape, lambda i: (0, 0)),
            pl.BlockSpec((1, 1), lambda i: (0, 0)),
        ],
        out_specs=pl.BlockSpec((1, eblk // 128, 128), lambda i: (i, 0, 0)),
        out_shape=jax.ShapeDtypeStruct((e // eblk, eblk // 128, 128),
                                       jnp.float32),
    )(S, fc1_b, fc2_W, fc2_b, fc3_W, fc3_b, fc4_W, fc4_b)


def kernel(x, edge_index, W1, b1, W2, b2,
           fc1_W, fc1_b, fc2_W, fc2_b, fc3_W, fc3_b, fc4_W, fc4_b):
    n, cin = x.shape
    e = edge_index.shape[1]
    nw = NC * NS

    ei = edge_index.astype(jnp.int32)
    src, dst = ei[0], ei[1]

    n_pad = -(-n // (NS * 8)) * (NS * 8)

    del nw
    zeros1 = jnp.zeros((n_pad,), jnp.float32)
    zeros128 = jnp.zeros((n_pad, W1.shape[1] // 2), jnp.float32)

    degp = _sc_degree(dst, zeros1, n_pad).T

    Y1p = _tc_y1(x, W1, degp, rblk=1000)
    A1 = _sc_aggregate(Y1p.reshape(2 * n, -1), src, dst, zeros128,
                       n_pad, W1.shape[1] // 2,
                       feat_split=True)

    Y2 = _tc_layer2(A1, Y1p, degp, b1.reshape(1, -1), W2, rblk=1000)
    A2 = _sc_aggregate(Y2, src, dst, zeros128, n_pad, W2.shape[1],
                       feat_split=False)

    fcA = fc1_W[:W2.shape[1]]
    fcB = fc1_W[W2.shape[1]:]
    zA, zB = _tc_z_proj(A2, Y2, degp, b2.reshape(1, -1), fcA, fcB, rblk=1000)

    S = _sc_decode(zA, zB, src, dst, fc1_W.shape[1])

    out = _tc_mlp(S, fc1_b.reshape(1, -1), fc2_W, fc2_b.reshape(1, -1),
                  fc3_W, fc3_b.reshape(1, -1), fc4_W,
                  fc4_b.reshape(1, -1), eblk=6400)
    return out.reshape(-1)


# speedup vs baseline: 1.2568x; 1.0757x over previous
import functools

import jax
import jax.numpy as jnp
from jax import lax
from jax.experimental import pallas as pl
from jax.experimental.pallas import tpu as pltpu
from jax.experimental.pallas import tpu_sc as plsc

NC = 2
NS = 16
K = 80


def _mesh():
    return plsc.VectorSubcoreMesh(core_axis_name="c", subcore_axis_name="s")


def _sc_degree(eif, zeros1, n_pad):
    half = eif.shape[0] // 2
    ept = half // (NC * NS)

    @functools.partial(
        pl.kernel,
        out_type=jax.ShapeDtypeStruct((NC * NS, n_pad), jnp.float32),
        mesh=_mesh(),
        compiler_params=pltpu.CompilerParams(needs_layout_passes=False),
        scratch_types=[
            pltpu.VMEM((ept,), jnp.int32),
            pltpu.VMEM((n_pad,), jnp.float32),
        ],
    )
    def k(ei_hbm, zero_h, out_hbm, idx_v, hist):
        c = lax.axis_index("c")
        s = lax.axis_index("s")
        w = c * NS + s
        pltpu.sync_copy(zero_h, hist)
        pltpu.sync_copy(ei_hbm.at[pl.ds(half + w * ept, ept)], idx_v)
        ones = jnp.ones((16,), jnp.float32)
        lanes = lax.iota(jnp.int32, 16)

        def body(j, _):
            idx = idx_v[pl.ds(j * 16, 16)]
            for m in range(16):
                plsc.addupdate_scatter(hist, [idx], ones, mask=lanes == m)
            return ()
        lax.fori_loop(0, ept // 16, body, ())
        pltpu.sync_copy(hist, out_hbm.at[w])

    return k(eif, zeros1)


RING = 3


def _sc_aggregate(table, eif, zeros_nd, n_pad, d, feat_split):
    e = eif.shape[0] // 2
    ept = e // NS if feat_split else e // (NC * NS)
    nch = ept // K
    coeff = table.shape[0] // 2 if feat_split else 0
    rows_per_tile = n_pad // NS
    R = RING

    @functools.partial(
        pl.kernel,
        out_type=jax.ShapeDtypeStruct((NC, n_pad, d), jnp.float32),
        mesh=_mesh(),
        compiler_params=pltpu.CompilerParams(needs_layout_passes=False),
        scratch_types=(
            [pltpu.VMEM((K,), jnp.int32) for _ in range(2 * R)]
            + [pltpu.VMEM((K, d), jnp.float32) for _ in range(R)]
            + [pltpu.VMEM_SHARED((n_pad, d), jnp.float32)]
            + [pltpu.SemaphoreType.DMA for _ in range(R)]
        ),
    )
    def k(tbl, ei_h, zero_h, out_hbm, *scr):
        igb = scr[0:R]
        dsb = scr[R:2 * R]
        rows = scr[2 * R:3 * R]
        acc = scr[3 * R]
        sems = scr[3 * R + 1:]
        c = lax.axis_index("c")
        s = lax.axis_index("s")
        rs = s * rows_per_tile
        eoff = (s if feat_split else c * NS + s) * ept
        off = c * coeff
        pltpu.sync_copy(zero_h.at[pl.ds(rs, rows_per_tile)],
                        acc.at[pl.ds(rs, rows_per_tile)])

        def load_idx(j, b):
            pltpu.sync_copy(ei_h.at[pl.ds(eoff + j * K, K)], igb[b])
            pltpu.sync_copy(ei_h.at[pl.ds(e + eoff + j * K, K)], dsb[b])
            if feat_split:
                for v in range(K // 16):
                    sl = pl.ds(v * 16, 16)
                    igb[b][sl] = igb[b][sl] + off

        for b in range(R):
            load_idx(b, b)
        plsc.subcore_barrier()

        for b in range(R):
            pltpu.async_copy(tbl.at[igb[b]], rows[b], sems[b])

        def step(j, b):
            pltpu.make_async_copy(tbl.at[igb[b]], rows[b], sems[b]).wait()
            pltpu.sync_copy(rows[b], acc.at[dsb[b]], add=True)

            def refill():
                load_idx(j + R, b)
                pltpu.async_copy(tbl.at[igb[b]], rows[b], sems[b])

            if isinstance(j, int):
                if j + R < nch:
                    refill()
            else:
                pl.when(j + R < nch)(refill)

        def body(p, _):
            for b in range(R):
                step(p * R + b, b)
            return ()
        lax.fori_loop(0, nch // R, body, ())
        for j in range((nch // R) * R, nch):
            step(j, j % R)
        plsc.subcore_barrier()
        pltpu.sync_copy(acc.at[pl.ds(rs, rows_per_tile)],
                        out_hbm.at[c, pl.ds(rs, rows_per_tile)])

    return k(table, eif, zeros_nd)


def _sc_decode(zA, zB, eif, d):
    e = eif.shape[0] // 2
    ept = e // (NC * NS)
    nch = ept // K

    @functools.partial(
        pl.kernel,
        out_type=jax.ShapeDtypeStruct((e, d), jnp.float32),
        mesh=_mesh(),
        scratch_types=(
            [pltpu.VMEM((ept,), jnp.int32) for _ in range(2)]
            + [pltpu.VMEM((K, d), jnp.float32) for _ in range(6)]
            + [pltpu.SemaphoreType.DMA for _ in range(6)]
        ),
    )
    def k(za, zb, ei_h, out_hbm, *scr):
        sgv, dgv = scr[0:2]
        abuf = scr[2:4]
        bbuf = scr[4:6]
        svs = scr[6:8]
        sas = scr[8:10]
        sbs = scr[10:12]
        sws = scr[12:14]
        c = lax.axis_index("c")
        s = lax.axis_index("s")
        base = (c * NS + s) * ept
        pltpu.sync_copy(ei_h.at[pl.ds(base, ept)], sgv)
        pltpu.sync_copy(ei_h.at[pl.ds(e + base, ept)], dgv)

        def fire(j, b):
            pltpu.async_copy(za.at[sgv.at[pl.ds(j * K, K)]], abuf[b], sas[b])
            pltpu.async_copy(zb.at[dgv.at[pl.ds(j * K, K)]], bbuf[b], sbs[b])

        for b in range(2):
            fire(b, b)

        nv = d // 16

        def step(j, b):
            pltpu.make_async_copy(za.at[sgv.at[pl.ds(0, K)]], abuf[b],
                                  sas[b]).wait()
            pltpu.make_async_copy(zb.at[dgv.at[pl.ds(0, K)]], bbuf[b],
                                  sbs[b]).wait()

            def drain_write():
                pltpu.make_async_copy(
                    svs[b], out_hbm.at[pl.ds(base, K)], sws[b]).wait()

            if isinstance(j, int):
                if j >= 2:
                    drain_write()
            else:
                pl.when(j >= 2)(drain_write)

            def add_row(r, _):
                for v in range(nv):
                    sl = pl.ds(v * 16, 16)
                    svs[b][r, sl] = abuf[b][r, sl] + bbuf[b][r, sl]
                return ()
            lax.fori_loop(0, K, add_row, ())
            pltpu.async_copy(svs[b], out_hbm.at[pl.ds(base + j * K, K)],
                             sws[b])

            def refill():
                fire(j + 2, b)

            if isinstance(j, int):
                if j + 2 < nch:
                    refill()
            else:
                pl.when(j + 2 < nch)(refill)

        def body(p, _):
            for b in range(2):
                step(p * 2 + b, b)
            return ()
        lax.fori_loop(0, nch // 2, body, ())
        for j in range((nch // 2) * 2, nch):
            step(j, j % 2)
        for b in range(2):
            pltpu.make_async_copy(svs[b], out_hbm.at[pl.ds(base, K)],
                                  sws[b]).wait()

    return k(zA, zB, eif)


def _dis_block(dp_ref, rblk):
    del rblk
    return lax.rsqrt(jnp.sum(dp_ref[...], axis=1) + 1.0)[:, None]


def _tc_y1(x, W1, degp, rblk):
    n, cin = x.shape
    cout = W1.shape[1]
    half = cout // 2

    def body(x_ref, w_ref, dp_ref, y_ref):
        dis = _dis_block(dp_ref, rblk)
        y = dis * jnp.dot(x_ref[...], w_ref[...],
                          preferred_element_type=jnp.float32)
        y_ref[0, ...] = y[:, :half]
        y_ref[1, ...] = y[:, half:]

    return pl.pallas_call(
        body,
        grid=(n // rblk,),
        in_specs=[
            pl.BlockSpec((rblk, cin), lambda i: (i, 0)),
            pl.BlockSpec((cin, cout), lambda i: (0, 0)),
            pl.BlockSpec((rblk, NC * NS), lambda i: (i, 0)),
        ],
        out_specs=pl.BlockSpec((2, rblk, half), lambda i: (0, i, 0)),
        out_shape=jax.ShapeDtypeStruct((2, n, half), jnp.float32),
    )(x, W1, degp)


def _tc_layer2(A1, Y1p, degp, b1, W2, rblk):
    n = Y1p.shape[1]
    d1 = 2 * Y1p.shape[2]
    d2 = W2.shape[1]

    def body(a_ref, y_ref, dp_ref, b_ref, w_ref, y2_ref):
        dis = _dis_block(dp_ref, rblk)
        agg = jnp.concatenate([a_ref[0], a_ref[1]], axis=-1)
        y = jnp.concatenate([y_ref[0], y_ref[1]], axis=-1)
        h = jnp.maximum(dis * (agg + y) + b_ref[...], 0.0)
        y2_ref[...] = dis * jnp.dot(h, w_ref[...],
                                    preferred_element_type=jnp.float32)

    return pl.pallas_call(
        body,
        grid=(n // rblk,),
        in_specs=[
            pl.BlockSpec((2, rblk, d1 // 2), lambda i: (0, i, 0)),
            pl.BlockSpec((2, rblk, d1 // 2), lambda i: (0, i, 0)),
            pl.BlockSpec((rblk, NC * NS), lambda i: (i, 0)),
            pl.BlockSpec((1, d1), lambda i: (0, 0)),
            pl.BlockSpec((d1, d2), lambda i: (0, 0)),
        ],
        out_specs=pl.BlockSpec((rblk, d2), lambda i: (i, 0)),
        out_shape=jax.ShapeDtypeStruct((n, d2), jnp.float32),
    )(A1, Y1p, degp, b1, W2)


def _tc_z_proj(A2, Y2, degp, b2, fcA, fcB, rblk):
    n, d2 = Y2.shape
    dp = fcA.shape[1]

    def body(a_ref, y_ref, dp_ref, b_ref, wa_ref, wb_ref, za_ref, zb_ref):
        dis = _dis_block(dp_ref, rblk)
        agg = a_ref[0] + a_ref[1]
        z = dis * (agg + y_ref[...]) + b_ref[...]
        za_ref[...] = jnp.dot(z, wa_ref[...], preferred_element_type=jnp.float32)
        zb_ref[...] = jnp.dot(z, wb_ref[...], preferred_element_type=jnp.float32)

    return pl.pallas_call(
        body,
        grid=(n // rblk,),
        in_specs=[
            pl.BlockSpec((2, rblk, d2), lambda i: (0, i, 0)),
            pl.BlockSpec((rblk, d2), lambda i: (i, 0)),
            pl.BlockSpec((rblk, NC * NS), lambda i: (i, 0)),
            pl.BlockSpec((1, d2), lambda i: (0, 0)),
            pl.BlockSpec((d2, dp), lambda i: (0, 0)),
            pl.BlockSpec((d2, dp), lambda i: (0, 0)),
        ],
        out_specs=[
            pl.BlockSpec((rblk, dp), lambda i: (i, 0)),
            pl.BlockSpec((rblk, dp), lambda i: (i, 0)),
        ],
        out_shape=[
            jax.ShapeDtypeStruct((n, dp), jnp.float32),
            jax.ShapeDtypeStruct((n, dp), jnp.float32),
        ],
    )(A2, Y2, degp, b2, fcA, fcB)


def _tc_mlp(S, fc1_b, fc2_W, fc2_b, fc3_W, fc3_b, fc4_W, fc4_b, eblk):
    e, dh = S.shape

    bf = jnp.bfloat16

    def body(s_ref, b1_ref, w2_ref, b2_ref, w3_ref, b3_ref, w4_ref, b4_ref,
             o_ref):
        v = jnp.maximum(s_ref[...] + b1_ref[...], 0.0)
        v = jnp.maximum(jnp.dot(v.astype(bf), w2_ref[...].astype(bf),
                                preferred_element_type=jnp.float32)
                        + b2_ref[...], 0.0)
        v = jnp.maximum(jnp.dot(v.astype(bf), w3_ref[...].astype(bf),
                                preferred_element_type=jnp.float32)
                        + b3_ref[...], 0.0)
        o = jnp.sum(v * w4_ref[...].reshape(1, -1), axis=1) + b4_ref[0, 0]
        o_ref[...] = o.reshape(1, eblk // 128, 128)

    return pl.pallas_call(
        body,
        grid=(e // eblk,),
        in_specs=[
            pl.BlockSpec((eblk, dh), lambda i: (i, 0)),
            pl.BlockSpec((1, dh), lambda i: (0, 0)),
            pl.BlockSpec(fc2_W.shape, lambda i: (0, 0)),
            pl.BlockSpec((1, fc2_W.shape[1]), lambda i: (0, 0)),
            pl.BlockSpec(fc3_W.shape, lambda i: (0, 0)),
            pl.BlockSpec((1, fc3_W.shape[1]), lambda i: (0, 0)),
            pl.BlockSpec(fc4_W.shape, lambda i: (0, 0)),
            pl.BlockSpec((1, 1), lambda i: (0, 0)),
        ],
        out_specs=pl.BlockSpec((1, eblk // 128, 128), lambda i: (i, 0, 0)),
        out_shape=jax.ShapeDtypeStruct((e // eblk, eblk // 128, 128),
                                       jnp.float32),
    )(S, fc1_b, fc2_W, fc2_b, fc3_W, fc3_b, fc4_W, fc4_b)


def kernel(x, edge_index, W1, b1, W2, b2,
           fc1_W, fc1_b, fc2_W, fc2_b, fc3_W, fc3_b, fc4_W, fc4_b):
    n, cin = x.shape
    e = edge_index.shape[1]
    nw = NC * NS

    eif = edge_index.astype(jnp.int32).reshape(-1)

    n_pad = -(-n // (NS * 8)) * (NS * 8)

    del nw
    zeros1 = jnp.zeros((n_pad,), jnp.float32)
    zeros128 = jnp.zeros((n_pad, W1.shape[1] // 2), jnp.float32)

    degp = _sc_degree(eif, zeros1, n_pad).T

    Y1p = _tc_y1(x, W1, degp, rblk=1000)
    A1 = _sc_aggregate(Y1p.reshape(2 * n, -1), eif, zeros128,
                       n_pad, W1.shape[1] // 2,
                       feat_split=True)

    Y2 = _tc_layer2(A1, Y1p, degp, b1.reshape(1, -1), W2, rblk=1000)
    A2 = _sc_aggregate(Y2, eif, zeros128, n_pad, W2.shape[1],
                       feat_split=False)

    fcA = fc1_W[:W2.shape[1]]
    fcB = fc1_W[W2.shape[1]:]
    zA, zB = _tc_z_proj(A2, Y2, degp, b2.reshape(1, -1), fcA, fcB, rblk=1000)

    S = _sc_decode(zA, zB, eif, fc1_W.shape[1])

    out = _tc_mlp(S, fc1_b.reshape(1, -1), fc2_W, fc2_b.reshape(1, -1),
                  fc3_W, fc3_b.reshape(1, -1), fc4_W,
                  fc4_b.reshape(1, -1), eblk=6400)
    return out.reshape(-1)


# speedup vs baseline: 1.2765x; 1.0157x over previous
import functools

import jax
import jax.numpy as jnp
from jax import lax
from jax.experimental import pallas as pl
from jax.experimental.pallas import tpu as pltpu
from jax.experimental.pallas import tpu_sc as plsc

NC = 2
NS = 16
K = 80


def _mesh():
    return plsc.VectorSubcoreMesh(core_axis_name="c", subcore_axis_name="s")


def _sc_degree(eif, zeros1, n_pad):
    half = eif.shape[0] // 2
    ept = half // (NC * NS)

    @functools.partial(
        pl.kernel,
        out_type=jax.ShapeDtypeStruct((NC * NS, n_pad), jnp.float32),
        mesh=_mesh(),
        compiler_params=pltpu.CompilerParams(needs_layout_passes=False),
        scratch_types=[
            pltpu.VMEM((ept,), jnp.int32),
            pltpu.VMEM((n_pad,), jnp.float32),
        ],
    )
    def k(ei_hbm, zero_h, out_hbm, idx_v, hist):
        c = lax.axis_index("c")
        s = lax.axis_index("s")
        w = c * NS + s
        pltpu.sync_copy(zero_h, hist)
        pltpu.sync_copy(ei_hbm.at[pl.ds(half + w * ept, ept)], idx_v)
        ones = jnp.ones((16,), jnp.float32)
        lanes = lax.iota(jnp.int32, 16)

        def body(j, _):
            idx = idx_v[pl.ds(j * 16, 16)]
            for m in range(16):
                plsc.addupdate_scatter(hist, [idx], ones, mask=lanes == m)
            return ()
        lax.fori_loop(0, ept // 16, body, ())
        pltpu.sync_copy(hist, out_hbm.at[w])

    return k(eif, zeros1)


RING = 3


def _sc_aggregate(table, eif, zeros_nd, n_pad, d, feat_split):
    e = eif.shape[0] // 2
    ept = e // NS if feat_split else e // (NC * NS)
    nch = ept // K
    coeff = table.shape[0] // 2 if feat_split else 0
    rows_per_tile = n_pad // NS
    R = RING

    @functools.partial(
        pl.kernel,
        out_type=jax.ShapeDtypeStruct((NC, n_pad, d), jnp.float32),
        mesh=_mesh(),
        compiler_params=pltpu.CompilerParams(needs_layout_passes=False),
        scratch_types=(
            [pltpu.VMEM((K,), jnp.int32) for _ in range(2 * R)]
            + [pltpu.VMEM((K, d), jnp.float32) for _ in range(R)]
            + [pltpu.VMEM_SHARED((n_pad, d), jnp.float32)]
            + [pltpu.SemaphoreType.DMA for _ in range(R)]
        ),
    )
    def k(tbl, ei_h, zero_h, out_hbm, *scr):
        igb = scr[0:R]
        dsb = scr[R:2 * R]
        rows = scr[2 * R:3 * R]
        acc = scr[3 * R]
        sems = scr[3 * R + 1:]
        c = lax.axis_index("c")
        s = lax.axis_index("s")
        rs = s * rows_per_tile
        eoff = (s if feat_split else c * NS + s) * ept
        off = c * coeff
        pltpu.sync_copy(zero_h.at[pl.ds(rs, rows_per_tile)],
                        acc.at[pl.ds(rs, rows_per_tile)])

        def load_idx(j, b):
            pltpu.sync_copy(ei_h.at[pl.ds(eoff + j * K, K)], igb[b])
            pltpu.sync_copy(ei_h.at[pl.ds(e + eoff + j * K, K)], dsb[b])
            if feat_split:
                for v in range(K // 16):
                    sl = pl.ds(v * 16, 16)
                    igb[b][sl] = igb[b][sl] + off

        for b in range(R):
            load_idx(b, b)
        plsc.subcore_barrier()

        for b in range(R):
            pltpu.async_copy(tbl.at[igb[b]], rows[b], sems[b])

        def step(j, b):
            pltpu.make_async_copy(tbl.at[igb[b]], rows[b], sems[b]).wait()
            pltpu.sync_copy(rows[b], acc.at[dsb[b]], add=True)

            def refill():
                load_idx(j + R, b)
                pltpu.async_copy(tbl.at[igb[b]], rows[b], sems[b])

            if isinstance(j, int):
                if j + R < nch:
                    refill()
            else:
                pl.when(j + R < nch)(refill)

        def body(p, _):
            for b in range(R):
                step(p * R + b, b)
            return ()
        lax.fori_loop(0, nch // R, body, ())
        for j in range((nch // R) * R, nch):
            step(j, j % R)
        plsc.subcore_barrier()
        pltpu.sync_copy(acc.at[pl.ds(rs, rows_per_tile)],
                        out_hbm.at[c, pl.ds(rs, rows_per_tile)])

    return k(table, eif, zeros_nd)


def _sc_decode(zA, zB, eif, d):
    e = eif.shape[0] // 2
    ept = e // (NC * NS)
    nch = ept // K

    @functools.partial(
        pl.kernel,
        out_type=jax.ShapeDtypeStruct((e, d), jnp.float32),
        mesh=_mesh(),
        scratch_types=(
            [pltpu.VMEM((ept,), jnp.int32) for _ in range(2)]
            + [pltpu.VMEM((K, d), jnp.float32) for _ in range(6)]
            + [pltpu.SemaphoreType.DMA for _ in range(6)]
        ),
    )
    def k(za, zb, ei_h, out_hbm, *scr):
        sgv, dgv = scr[0:2]
        abuf = scr[2:4]
        bbuf = scr[4:6]
        svs = scr[6:8]
        sas = scr[8:10]
        sbs = scr[10:12]
        sws = scr[12:14]
        c = lax.axis_index("c")
        s = lax.axis_index("s")
        base = (c * NS + s) * ept
        pltpu.sync_copy(ei_h.at[pl.ds(base, ept)], sgv)
        pltpu.sync_copy(ei_h.at[pl.ds(e + base, ept)], dgv)

        def fire(j, b):
            pltpu.async_copy(za.at[sgv.at[pl.ds(j * K, K)]], abuf[b], sas[b])
            pltpu.async_copy(zb.at[dgv.at[pl.ds(j * K, K)]], bbuf[b], sbs[b])

        for b in range(2):
            fire(b, b)

        nv = d // 16

        def step(j, b):
            pltpu.make_async_copy(za.at[sgv.at[pl.ds(0, K)]], abuf[b],
                                  sas[b]).wait()
            pltpu.make_async_copy(zb.at[dgv.at[pl.ds(0, K)]], bbuf[b],
                                  sbs[b]).wait()

            def drain_write():
                pltpu.make_async_copy(
                    svs[b], out_hbm.at[pl.ds(base, K)], sws[b]).wait()

            if isinstance(j, int):
                if j >= 2:
                    drain_write()
            else:
                pl.when(j >= 2)(drain_write)

            def add_row(r, _):
                for v in range(nv):
                    sl = pl.ds(v * 16, 16)
                    svs[b][r, sl] = abuf[b][r, sl] + bbuf[b][r, sl]
                return ()
            lax.fori_loop(0, K, add_row, ())
            pltpu.async_copy(svs[b], out_hbm.at[pl.ds(base + j * K, K)],
                             sws[b])

            def refill():
                fire(j + 2, b)

            if isinstance(j, int):
                if j + 2 < nch:
                    refill()
            else:
                pl.when(j + 2 < nch)(refill)

        def body(p, _):
            for b in range(2):
                step(p * 2 + b, b)
            return ()
        lax.fori_loop(0, nch // 2, body, ())
        for j in range((nch // 2) * 2, nch):
            step(j, j % 2)
        for b in range(2):
            pltpu.make_async_copy(svs[b], out_hbm.at[pl.ds(base, K)],
                                  sws[b]).wait()

    return k(zA, zB, eif)


def _dis_block(dp_ref, rblk):
    del rblk
    return lax.rsqrt(jnp.sum(dp_ref[...], axis=1) + 1.0)[:, None]


def _tc_y1(x, W1, degp, rblk):
    n, cin = x.shape
    cout = W1.shape[1]
    half = cout // 2

    def body(x_ref, w_ref, dp_ref, y_ref):
        dis = _dis_block(dp_ref, rblk)
        y = dis * jnp.dot(x_ref[...], w_ref[...],
                          preferred_element_type=jnp.float32)
        y_ref[0, ...] = y[:, :half]
        y_ref[1, ...] = y[:, half:]

    return pl.pallas_call(
        body,
        grid=(n // rblk,),
        in_specs=[
            pl.BlockSpec((rblk, cin), lambda i: (i, 0)),
            pl.BlockSpec((cin, cout), lambda i: (0, 0)),
            pl.BlockSpec((rblk, NC * NS), lambda i: (i, 0)),
        ],
        out_specs=pl.BlockSpec((2, rblk, half), lambda i: (0, i, 0)),
        out_shape=jax.ShapeDtypeStruct((2, n, half), jnp.float32),
    )(x, W1, degp)


def _tc_layer2(A1, Y1p, degp, b1, W2, rblk):
    n = Y1p.shape[1]
    d1 = 2 * Y1p.shape[2]
    d2 = W2.shape[1]

    def body(a_ref, y_ref, dp_ref, b_ref, w_ref, y2_ref):
        dis = _dis_block(dp_ref, rblk)
        agg = jnp.concatenate([a_ref[0], a_ref[1]], axis=-1)
        y = jnp.concatenate([y_ref[0], y_ref[1]], axis=-1)
        h = jnp.maximum(dis * (agg + y) + b_ref[...], 0.0)
        y2_ref[...] = dis * jnp.dot(h, w_ref[...],
                                    preferred_element_type=jnp.float32)

    return pl.pallas_call(
        body,
        grid=(n // rblk,),
        in_specs=[
            pl.BlockSpec((2, rblk, d1 // 2), lambda i: (0, i, 0)),
            pl.BlockSpec((2, rblk, d1 // 2), lambda i: (0, i, 0)),
            pl.BlockSpec((rblk, NC * NS), lambda i: (i, 0)),
            pl.BlockSpec((1, d1), lambda i: (0, 0)),
            pl.BlockSpec((d1, d2), lambda i: (0, 0)),
        ],
        out_specs=pl.BlockSpec((rblk, d2), lambda i: (i, 0)),
        out_shape=jax.ShapeDtypeStruct((n, d2), jnp.float32),
    )(A1, Y1p, degp, b1, W2)


def _tc_z_proj(A2, Y2, degp, b2, fcA, fcB, rblk):
    n, d2 = Y2.shape
    dp = fcA.shape[1]

    def body(a_ref, y_ref, dp_ref, b_ref, wa_ref, wb_ref, za_ref, zb_ref):
        dis = _dis_block(dp_ref, rblk)
        agg = a_ref[0] + a_ref[1]
        z = dis * (agg + y_ref[...]) + b_ref[...]
        za_ref[...] = jnp.dot(z, wa_ref[...], preferred_element_type=jnp.float32)
        zb_ref[...] = jnp.dot(z, wb_ref[...], preferred_element_type=jnp.float32)

    return pl.pallas_call(
        body,
        grid=(n // rblk,),
        in_specs=[
            pl.BlockSpec((2, rblk, d2), lambda i: (0, i, 0)),
            pl.BlockSpec((rblk, d2), lambda i: (i, 0)),
            pl.BlockSpec((rblk, NC * NS), lambda i: (i, 0)),
            pl.BlockSpec((1, d2), lambda i: (0, 0)),
            pl.BlockSpec((d2, dp), lambda i: (0, 0)),
            pl.BlockSpec((d2, dp), lambda i: (0, 0)),
        ],
        out_specs=[
            pl.BlockSpec((rblk, dp), lambda i: (i, 0)),
            pl.BlockSpec((rblk, dp), lambda i: (i, 0)),
        ],
        out_shape=[
            jax.ShapeDtypeStruct((n, dp), jnp.float32),
            jax.ShapeDtypeStruct((n, dp), jnp.float32),
        ],
    )(A2, Y2, degp, b2, fcA, fcB)


def _tc_mlp(S, fc1_b, fc2_W, fc2_b, fc3_W, fc3_b, fc4_W, fc4_b, eblk):
    e, dh = S.shape

    bf = jnp.bfloat16

    def body(s_ref, b1_ref, w2_ref, b2_ref, w3_ref, b3_ref, w4_ref, b4_ref,
             o_ref):
        v = jnp.maximum(s_ref[...] + b1_ref[...], 0.0)
        v = jnp.maximum(jnp.dot(v.astype(bf), w2_ref[...].astype(bf),
                                preferred_element_type=jnp.float32)
                        + b2_ref[...], 0.0)
        v = jnp.maximum(jnp.dot(v.astype(bf), w3_ref[...].astype(bf),
                                preferred_element_type=jnp.float32)
                        + b3_ref[...], 0.0)
        o = jnp.sum(v * w4_ref[...].reshape(1, -1), axis=1) + b4_ref[0, 0]
        o_ref[...] = o.reshape(1, eblk // 128, 128)

    return pl.pallas_call(
        body,
        grid=(e // eblk,),
        in_specs=[
            pl.BlockSpec((eblk, dh), lambda i: (i, 0)),
            pl.BlockSpec((1, dh), lambda i: (0, 0)),
            pl.BlockSpec(fc2_W.shape, lambda i: (0, 0)),
            pl.BlockSpec((1, fc2_W.shape[1]), lambda i: (0, 0)),
            pl.BlockSpec(fc3_W.shape, lambda i: (0, 0)),
            pl.BlockSpec((1, fc3_W.shape[1]), lambda i: (0, 0)),
            pl.BlockSpec(fc4_W.shape, lambda i: (0, 0)),
            pl.BlockSpec((1, 1), lambda i: (0, 0)),
        ],
        out_specs=pl.BlockSpec((1, eblk // 128, 128), lambda i: (i, 0, 0)),
        out_shape=jax.ShapeDtypeStruct((e // eblk, eblk // 128, 128),
                                       jnp.float32),
    )(S, fc1_b, fc2_W, fc2_b, fc3_W, fc3_b, fc4_W, fc4_b)


def kernel(x, edge_index, W1, b1, W2, b2,
           fc1_W, fc1_b, fc2_W, fc2_b, fc3_W, fc3_b, fc4_W, fc4_b):
    n, cin = x.shape
    e = edge_index.shape[1]
    nw = NC * NS

    eif = edge_index.astype(jnp.int32).reshape(-1)

    n_pad = -(-n // (NS * 8)) * (NS * 8)

    del nw
    zeros1 = jnp.zeros((n_pad,), jnp.float32)
    zeros128 = jnp.zeros((n_pad, W1.shape[1] // 2), jnp.float32)

    degp = _sc_degree(eif, zeros1, n_pad).T

    Y1p = _tc_y1(x, W1, degp, rblk=1000)
    A1 = _sc_aggregate(Y1p.reshape(2 * n, -1), eif, zeros128,
                       n_pad, W1.shape[1] // 2,
                       feat_split=True)

    Y2 = _tc_layer2(A1, Y1p, degp, b1.reshape(1, -1), W2, rblk=1000)
    A2 = _sc_aggregate(Y2, eif, zeros128, n_pad, W2.shape[1],
                       feat_split=False)

    fcA = fc1_W[:W2.shape[1]]
    fcB = fc1_W[W2.shape[1]:]
    zA, zB = _tc_z_proj(A2, Y2, degp, b2.reshape(1, -1), fcA, fcB, rblk=1000)

    S = _sc_decode(zA, zB, eif, fc1_W.shape[1])

    out = _tc_mlp(S, fc1_b.reshape(1, -1), fc2_W, fc2_b.reshape(1, -1),
                  fc3_W, fc3_b.reshape(1, -1), fc4_W,
                  fc4_b.reshape(1, -1), eblk=12800)
    return out.reshape(-1)


# speedup vs baseline: 1.2783x; 1.0014x over previous
import functools

import jax
import jax.numpy as jnp
from jax import lax
from jax.experimental import pallas as pl
from jax.experimental.pallas import tpu as pltpu
from jax.experimental.pallas import tpu_sc as plsc

NC = 2
NS = 16
K = 80


def _mesh():
    return plsc.VectorSubcoreMesh(core_axis_name="c", subcore_axis_name="s")


def _sc_degree(eif, zeros1, n_pad):
    half = eif.shape[0] // 2
    ept = half // (NC * NS)

    @functools.partial(
        pl.kernel,
        out_type=jax.ShapeDtypeStruct((NC * NS, n_pad), jnp.float32),
        mesh=_mesh(),
        compiler_params=pltpu.CompilerParams(needs_layout_passes=False),
        scratch_types=[
            pltpu.VMEM((ept,), jnp.int32),
            pltpu.VMEM((n_pad,), jnp.float32),
        ],
    )
    def k(ei_hbm, zero_h, out_hbm, idx_v, hist):
        c = lax.axis_index("c")
        s = lax.axis_index("s")
        w = c * NS + s
        pltpu.sync_copy(zero_h, hist)
        pltpu.sync_copy(ei_hbm.at[pl.ds(half + w * ept, ept)], idx_v)
        ones = jnp.ones((16,), jnp.float32)
        lanes = lax.iota(jnp.int32, 16)

        def body(j, _):
            idx = idx_v[pl.ds(j * 16, 16)]
            for m in range(16):
                plsc.addupdate_scatter(hist, [idx], ones, mask=lanes == m)
            return ()
        lax.fori_loop(0, ept // 16, body, ())
        pltpu.sync_copy(hist, out_hbm.at[w])

    return k(eif, zeros1)


RING = 3


def _sc_aggregate(table, eif, zeros_nd, n_pad, d, feat_split):
    e = eif.shape[0] // 2
    ept = e // NS if feat_split else e // (NC * NS)
    nch = ept // K
    coeff = table.shape[0] // 2 if feat_split else 0
    rows_per_tile = n_pad // NS
    R = RING

    @functools.partial(
        pl.kernel,
        out_type=jax.ShapeDtypeStruct((NC, n_pad, d), jnp.float32),
        mesh=_mesh(),
        compiler_params=pltpu.CompilerParams(needs_layout_passes=False),
        scratch_types=(
            [pltpu.VMEM((K,), jnp.int32) for _ in range(2 * R)]
            + [pltpu.VMEM((K, d), jnp.float32) for _ in range(R)]
            + [pltpu.VMEM_SHARED((n_pad, d), jnp.float32)]
            + [pltpu.SemaphoreType.DMA for _ in range(R)]
        ),
    )
    def k(tbl, ei_h, zero_h, out_hbm, *scr):
        igb = scr[0:R]
        dsb = scr[R:2 * R]
        rows = scr[2 * R:3 * R]
        acc = scr[3 * R]
        sems = scr[3 * R + 1:]
        c = lax.axis_index("c")
        s = lax.axis_index("s")
        rs = s * rows_per_tile
        eoff = (s if feat_split else c * NS + s) * ept
        off = c * coeff
        pltpu.sync_copy(zero_h.at[pl.ds(rs, rows_per_tile)],
                        acc.at[pl.ds(rs, rows_per_tile)])

        def load_idx(j, b):
            pltpu.sync_copy(ei_h.at[pl.ds(eoff + j * K, K)], igb[b])
            pltpu.sync_copy(ei_h.at[pl.ds(e + eoff + j * K, K)], dsb[b])
            if feat_split:
                for v in range(K // 16):
                    sl = pl.ds(v * 16, 16)
                    igb[b][sl] = igb[b][sl] + off

        for b in range(R):
            load_idx(b, b)
        plsc.subcore_barrier()

        for b in range(R):
            pltpu.async_copy(tbl.at[igb[b]], rows[b], sems[b])

        def step(j, b):
            pltpu.make_async_copy(tbl.at[igb[b]], rows[b], sems[b]).wait()
            pltpu.sync_copy(rows[b], acc.at[dsb[b]], add=True)

            def refill():
                load_idx(j + R, b)
                pltpu.async_copy(tbl.at[igb[b]], rows[b], sems[b])

            if isinstance(j, int):
                if j + R < nch:
                    refill()
            else:
                pl.when(j + R < nch)(refill)

        def body(p, _):
            for b in range(R):
                step(p * R + b, b)
            return ()
        lax.fori_loop(0, nch // R, body, ())
        for j in range((nch // R) * R, nch):
            step(j, j % R)
        plsc.subcore_barrier()
        pltpu.sync_copy(acc.at[pl.ds(rs, rows_per_tile)],
                        out_hbm.at[c, pl.ds(rs, rows_per_tile)])

    return k(table, eif, zeros_nd)


def _sc_decode(zA, zB, eif, d):
    e = eif.shape[0] // 2
    ept = e // (NC * NS)
    nch = ept // K

    @functools.partial(
        pl.kernel,
        out_type=jax.ShapeDtypeStruct((e, d), jnp.float32),
        mesh=_mesh(),
        scratch_types=(
            [pltpu.VMEM((ept,), jnp.int32) for _ in range(2)]
            + [pltpu.VMEM((K, d), jnp.float32) for _ in range(6)]
            + [pltpu.SemaphoreType.DMA for _ in range(6)]
        ),
    )
    def k(za, zb, ei_h, out_hbm, *scr):
        sgv, dgv = scr[0:2]
        abuf = scr[2:4]
        bbuf = scr[4:6]
        svs = scr[6:8]
        sas = scr[8:10]
        sbs = scr[10:12]
        sws = scr[12:14]
        c = lax.axis_index("c")
        s = lax.axis_index("s")
        base = (c * NS + s) * ept
        pltpu.sync_copy(ei_h.at[pl.ds(base, ept)], sgv)
        pltpu.sync_copy(ei_h.at[pl.ds(e + base, ept)], dgv)

        def fire(j, b):
            pltpu.async_copy(za.at[sgv.at[pl.ds(j * K, K)]], abuf[b], sas[b])
            pltpu.async_copy(zb.at[dgv.at[pl.ds(j * K, K)]], bbuf[b], sbs[b])

        for b in range(2):
            fire(b, b)

        nv = d // 16

        def step(j, b):
            pltpu.make_async_copy(za.at[sgv.at[pl.ds(0, K)]], abuf[b],
                                  sas[b]).wait()
            pltpu.make_async_copy(zb.at[dgv.at[pl.ds(0, K)]], bbuf[b],
                                  sbs[b]).wait()

            def drain_write():
                pltpu.make_async_copy(
                    svs[b], out_hbm.at[pl.ds(base, K)], sws[b]).wait()

            if isinstance(j, int):
                if j >= 2:
                    drain_write()
            else:
                pl.when(j >= 2)(drain_write)

            def add_row(r, _):
                for v in range(nv):
                    sl = pl.ds(v * 16, 16)
                    svs[b][r, sl] = abuf[b][r, sl] + bbuf[b][r, sl]
                return ()
            lax.fori_loop(0, K, add_row, ())
            pltpu.async_copy(svs[b], out_hbm.at[pl.ds(base + j * K, K)],
                             sws[b])

            def refill():
                fire(j + 2, b)

            if isinstance(j, int):
                if j + 2 < nch:
                    refill()
            else:
                pl.when(j + 2 < nch)(refill)

        def body(p, _):
            for b in range(2):
                step(p * 2 + b, b)
            return ()
        lax.fori_loop(0, nch // 2, body, ())
        for j in range((nch // 2) * 2, nch):
            step(j, j % 2)
        for b in range(2):
            pltpu.make_async_copy(svs[b], out_hbm.at[pl.ds(base, K)],
                                  sws[b]).wait()

    return k(zA, zB, eif)


def _dis_block(dp_ref, rblk):
    del rblk
    return lax.rsqrt(jnp.sum(dp_ref[...], axis=1) + 1.0)[:, None]


def _tc_y1(x, W1, degp, rblk):
    n, cin = x.shape
    cout = W1.shape[1]
    half = cout // 2

    def body(x_ref, w_ref, dp_ref, y_ref):
        dis = _dis_block(dp_ref, rblk)
        y = dis * jnp.dot(x_ref[...], w_ref[...],
                          preferred_element_type=jnp.float32)
        y_ref[0, ...] = y[:, :half]
        y_ref[1, ...] = y[:, half:]

    return pl.pallas_call(
        body,
        grid=(n // rblk,),
        in_specs=[
            pl.BlockSpec((rblk, cin), lambda i: (i, 0)),
            pl.BlockSpec((cin, cout), lambda i: (0, 0)),
            pl.BlockSpec((rblk, NC * NS), lambda i: (i, 0)),
        ],
        out_specs=pl.BlockSpec((2, rblk, half), lambda i: (0, i, 0)),
        out_shape=jax.ShapeDtypeStruct((2, n, half), jnp.float32),
    )(x, W1, degp)


def _tc_layer2(A1, Y1p, degp, b1, W2, rblk):
    n = Y1p.shape[1]
    d1 = 2 * Y1p.shape[2]
    d2 = W2.shape[1]

    def body(a_ref, y_ref, dp_ref, b_ref, w_ref, y2_ref):
        dis = _dis_block(dp_ref, rblk)
        agg = jnp.concatenate([a_ref[0], a_ref[1]], axis=-1)
        y = jnp.concatenate([y_ref[0], y_ref[1]], axis=-1)
        h = jnp.maximum(dis * (agg + y) + b_ref[...], 0.0)
        y2_ref[...] = dis * jnp.dot(h, w_ref[...],
                                    preferred_element_type=jnp.float32)

    return pl.pallas_call(
        body,
        grid=(n // rblk,),
        in_specs=[
            pl.BlockSpec((2, rblk, d1 // 2), lambda i: (0, i, 0)),
            pl.BlockSpec((2, rblk, d1 // 2), lambda i: (0, i, 0)),
            pl.BlockSpec((rblk, NC * NS), lambda i: (i, 0)),
            pl.BlockSpec((1, d1), lambda i: (0, 0)),
            pl.BlockSpec((d1, d2), lambda i: (0, 0)),
        ],
        out_specs=pl.BlockSpec((rblk, d2), lambda i: (i, 0)),
        out_shape=jax.ShapeDtypeStruct((n, d2), jnp.float32),
    )(A1, Y1p, degp, b1, W2)


def _tc_z_proj(A2, Y2, degp, b2, fcA, fcB, rblk):
    n, d2 = Y2.shape
    dp = fcA.shape[1]

    def body(a_ref, y_ref, dp_ref, b_ref, wa_ref, wb_ref, za_ref, zb_ref):
        dis = _dis_block(dp_ref, rblk)
        agg = a_ref[0] + a_ref[1]
        z = dis * (agg + y_ref[...]) + b_ref[...]
        za_ref[...] = jnp.dot(z, wa_ref[...], preferred_element_type=jnp.float32)
        zb_ref[...] = jnp.dot(z, wb_ref[...], preferred_element_type=jnp.float32)

    return pl.pallas_call(
        body,
        grid=(n // rblk,),
        in_specs=[
            pl.BlockSpec((2, rblk, d2), lambda i: (0, i, 0)),
            pl.BlockSpec((rblk, d2), lambda i: (i, 0)),
            pl.BlockSpec((rblk, NC * NS), lambda i: (i, 0)),
            pl.BlockSpec((1, d2), lambda i: (0, 0)),
            pl.BlockSpec((d2, dp), lambda i: (0, 0)),
            pl.BlockSpec((d2, dp), lambda i: (0, 0)),
        ],
        out_specs=[
            pl.BlockSpec((rblk, dp), lambda i: (i, 0)),
            pl.BlockSpec((rblk, dp), lambda i: (i, 0)),
        ],
        out_shape=[
            jax.ShapeDtypeStruct((n, dp), jnp.float32),
            jax.ShapeDtypeStruct((n, dp), jnp.float32),
        ],
    )(A2, Y2, degp, b2, fcA, fcB)


def _tc_mlp(S, fc1_b, fc2_W, fc2_b, fc3_W, fc3_b, fc4_W, fc4_b, eblk):
    e, dh = S.shape

    bf = jnp.bfloat16

    def body(s_ref, b1_ref, w2_ref, b2_ref, w3_ref, b3_ref, w4_ref, b4_ref,
             o_ref):
        v = jnp.maximum(s_ref[...] + b1_ref[...], 0.0)
        v = jnp.maximum(jnp.dot(v.astype(bf), w2_ref[...].astype(bf),
                                preferred_element_type=jnp.float32)
                        + b2_ref[...], 0.0)
        v = jnp.maximum(jnp.dot(v.astype(bf), w3_ref[...].astype(bf),
                                preferred_element_type=jnp.float32)
                        + b3_ref[...], 0.0)
        o = jnp.sum(v * w4_ref[...].reshape(1, -1), axis=1) + b4_ref[0, 0]
        o_ref[...] = o.reshape(1, eblk // 128, 128)

    return pl.pallas_call(
        body,
        grid=(e // eblk,),
        in_specs=[
            pl.BlockSpec((eblk, dh), lambda i: (i, 0)),
            pl.BlockSpec((1, dh), lambda i: (0, 0)),
            pl.BlockSpec(fc2_W.shape, lambda i: (0, 0)),
            pl.BlockSpec((1, fc2_W.shape[1]), lambda i: (0, 0)),
            pl.BlockSpec(fc3_W.shape, lambda i: (0, 0)),
            pl.BlockSpec((1, fc3_W.shape[1]), lambda i: (0, 0)),
            pl.BlockSpec(fc4_W.shape, lambda i: (0, 0)),
            pl.BlockSpec((1, 1), lambda i: (0, 0)),
        ],
        out_specs=pl.BlockSpec((1, eblk // 128, 128), lambda i: (i, 0, 0)),
        out_shape=jax.ShapeDtypeStruct((e // eblk, eblk // 128, 128),
                                       jnp.float32),
    )(S, fc1_b, fc2_W, fc2_b, fc3_W, fc3_b, fc4_W, fc4_b)


def kernel(x, edge_index, W1, b1, W2, b2,
           fc1_W, fc1_b, fc2_W, fc2_b, fc3_W, fc3_b, fc4_W, fc4_b):
    n, cin = x.shape
    e = edge_index.shape[1]
    nw = NC * NS

    eif = edge_index.astype(jnp.int32).reshape(-1)

    n_pad = -(-n // (NS * 8)) * (NS * 8)

    del nw
    zeros1 = jnp.zeros((n_pad,), jnp.float32)
    zeros128 = jnp.zeros((n_pad, W1.shape[1] // 2), jnp.float32)

    degp = _sc_degree(eif, zeros1, n_pad).T

    Y1p = _tc_y1(x, W1, degp, rblk=1000)
    A1 = _sc_aggregate(Y1p.reshape(2 * n, -1), eif, zeros128,
                       n_pad, W1.shape[1] // 2,
                       feat_split=True)

    Y2 = _tc_layer2(A1, Y1p, degp, b1.reshape(1, -1), W2, rblk=1000)
    A2 = _sc_aggregate(Y2, eif, zeros128, n_pad, W2.shape[1],
                       feat_split=False)

    fcA = fc1_W[:W2.shape[1]]
    fcB = fc1_W[W2.shape[1]:]
    zA, zB = _tc_z_proj(A2, Y2, degp, b2.reshape(1, -1), fcA, fcB, rblk=1000)

    S = _sc_decode(zA, zB, eif, fc1_W.shape[1])

    out = _tc_mlp(S, fc1_b.reshape(1, -1), fc2_W, fc2_b.reshape(1, -1),
                  fc3_W, fc3_b.reshape(1, -1), fc4_W,
                  fc4_b.reshape(1, -1), eblk=16000)
    return out.reshape(-1)
